# Initial kernel scaffold; baseline (speedup 1.0000x reference)
import jax
import jax.numpy as jnp
from jax.experimental import pallas as pl


def kernel(h, pos, W1a, b1a, W1b, b1b, W2a, b2a, W2b, b2b, Wc, bc, batch):
    raise NotImplementedError("write your pallas kernel here")


# speedup vs baseline: 2.3569x; 2.3569x over previous
import functools

import jax
import jax.numpy as jnp
from jax.experimental import pallas as pl
from jax.experimental.pallas import tpu as pltpu

_HID = 32
_NG = 8
_R2 = 625.0
_BI = 256
_BJ = 512
_JCH = 128
_NEG = -jnp.inf


def _accum_chunks(in_ch, relu_in, bi, bjr, pos_i, posT_j, fT_j, Wa, WapT, ba_c,
                  Wb, bb_c, acc):
    si = jnp.sum(pos_i * pos_i, axis=1, keepdims=True)
    Wa_h = Wa[:in_ch]
    bi_f = bi.astype(jnp.float32)
    n_i = pos_i.shape[0]
    wpb = WapT.astype(jnp.bfloat16).astype(jnp.float32)
    xi = pos_i[:, 0:1]
    yi = pos_i[:, 1:2]
    sh3 = (_HID, n_i, _JCH)
    wx3 = jax.lax.broadcast_in_dim(
        jnp.broadcast_to(wpb[:, 0:1], (_HID, _JCH)), sh3, (0, 2))
    wy3 = jax.lax.broadcast_in_dim(
        jnp.broadcast_to(wpb[:, 1:2], (_HID, _JCH)), sh3, (0, 2))
    for c in range(_BJ // _JCH):
        ls = slice(c * _JCH, (c + 1) * _JCH)
        pjt = posT_j[:, ls]
        fjt = fT_j[:, ls]
        if relu_in:
            fjt = jax.nn.relu(fjt)
        bj = bjr[:, ls].astype(jnp.float32)
        sj = jnp.sum(pjt * pjt, axis=0, keepdims=True)
        cross = jax.lax.dot_general(
            pos_i, pjt, (((1,), (0,)), ((), ())),
            preferred_element_type=jnp.float32)
        d2 = si + sj - 2.0 * cross
        pen = jnp.where((d2 <= _R2) & (bi_f == bj), 0.0, _NEG)
        ajh = jax.lax.dot_general(Wa_h, fjt, (((0,), (0,)), ((), ())),
                                  preferred_element_type=jnp.float32)
        ajh = ajh + ba_c
        dx = (pjt[0:1, :] - xi).astype(jnp.bfloat16).astype(jnp.float32)
        dy = (pjt[1:2, :] - yi).astype(jnp.bfloat16).astype(jnp.float32)
        pre3 = (jax.lax.broadcast_in_dim(ajh, sh3, (0, 2))
                + wx3 * jax.lax.broadcast_in_dim(dx, sh3, (1, 2))
                + wy3 * jax.lax.broadcast_in_dim(dy, sh3, (1, 2)))
        zT2 = jax.nn.relu(pre3).reshape(_HID, n_i * _JCH)
        mT = jax.lax.dot_general(Wb, zT2, (((0,), (0,)), ((), ())),
                                 preferred_element_type=jnp.float32) + bb_c
        mm = (mT.reshape(_HID, n_i, _JCH)
              + jax.lax.broadcast_in_dim(pen, (_HID, n_i, _JCH), (1, 2)))
        acc = jnp.maximum(acc, jnp.max(mm, axis=2))
    return acc


def _overlap(bi, bjr):
    return (jnp.min(bi) <= jnp.max(bjr)) & (jnp.min(bjr) <= jnp.max(bi))


def _l1_body(bi_ref, bjr_ref, pos_i_ref, posT_ref, hT_ref, Wa_ref, WapT_ref,
             ba_ref, Wb_ref, bb_ref, out_ref):
    j = pl.program_id(1)

    @pl.when(j == 0)
    def _():
        out_ref[...] = jnp.full_like(out_ref, _NEG)

    bi = bi_ref[...]
    bjr = bjr_ref[...]

    @pl.when(_overlap(bi, bjr))
    def _():
        out_ref[...] = _accum_chunks(
            3, False, bi, bjr, pos_i_ref[...], posT_ref[...], hT_ref[...],
            Wa_ref[...], WapT_ref[...], ba_ref[...], Wb_ref[...], bb_ref[...],
            out_ref[...])


def _l2_body(bi_ref, bjr_ref, bir_ref, pos_i_ref, posT_ref, xT_ref, Wa_ref,
             WapT_ref, ba_ref, Wb_ref, bb_ref, Wc_ref, bc_ref, out_ref,
             acc_ref, g_ref):
    i = pl.program_id(0)
    j = pl.program_id(1)
    nj = pl.num_programs(1)

    @pl.when(j == 0)
    def _():
        acc_ref[...] = jnp.full_like(acc_ref, _NEG)

    @pl.when((i == 0) & (j == 0))
    def _():
        g_ref[...] = jnp.full_like(g_ref, _NEG)

    bi = bi_ref[...]
    bjr = bjr_ref[...]

    @pl.when(_overlap(bi, bjr))
    def _():
        acc_ref[...] = _accum_chunks(
            _HID, True, bi, bjr, pos_i_ref[...], posT_ref[...], xT_ref[...],
            Wa_ref[...], WapT_ref[...], ba_ref[...], Wb_ref[...], bb_ref[...],
            acc_ref[...])

    @pl.when(j == nj - 1)
    def _():
        r = jax.nn.relu(acc_ref[...])
        bir = bir_ref[...]
        reds = jnp.concatenate(
            [jnp.max(jnp.where(bir == g, r, _NEG), axis=1, keepdims=True)
             for g in range(_NG)], axis=1)
        g_new = jnp.maximum(g_ref[...], reds)
        g_ref[...] = g_new
        out_ref[...] = (jax.lax.dot_general(
            g_new, Wc_ref[...], (((0,), (0,)), ((), ())),
            preferred_element_type=jnp.float32) + bc_ref[...])


def kernel(h, pos, W1a, b1a, W1b, b1b, W2a, b2a, W2b, b2b, Wc, bc, batch):
    n = h.shape[0]
    n_pad = max(-(-n // _BJ) * _BJ, _BJ)
    pad = n_pad - n

    hT = jnp.pad(h, ((0, pad), (0, 0))).T
    posp = jnp.pad(pos, ((0, pad), (0, 0)), constant_values=1e6)
    batchp = jnp.pad(batch.astype(jnp.int32), (0, pad), constant_values=-1)
    b_col = batchp.reshape(-1, 1)
    b_row = batchp.reshape(1, -1)
    posT = posp.T
    b1a_c = b1a.reshape(-1, 1)
    w1pT = W1a[3:].T
    b1b_c = b1b.reshape(-1, 1)
    b2a_c = b2a.reshape(-1, 1)
    w2pT = W2a[_HID:].T
    b2b_c = b2b.reshape(-1, 1)
    bc2 = bc.reshape(1, -1)

    ni = n_pad // _BI
    nj = n_pad // _BJ
    ispec = pl.BlockSpec((_BI, 1), lambda i, j: (i, 0))
    jspec_row = pl.BlockSpec((1, _BJ), lambda i, j: (0, j))
    ispec_row = pl.BlockSpec((1, _BI), lambda i, j: (0, i))
    pi_spec = pl.BlockSpec((_BI, 2), lambda i, j: (i, 0))
    pT_spec = pl.BlockSpec((2, _BJ), lambda i, j: (0, j))
    full = lambda r, c: pl.BlockSpec((r, c), lambda i, j: (0, 0))

    x1T = pl.pallas_call(
        _l1_body,
        grid=(ni, nj),
        in_specs=[ispec, jspec_row, pi_spec, pT_spec,
                  pl.BlockSpec((3, _BJ), lambda i, j: (0, j)),
                  full(5, _HID), full(_HID, 2), full(_HID, 1),
                  full(_HID, _HID), full(_HID, 1)],
        out_specs=pl.BlockSpec((_HID, _BI), lambda i, j: (0, i)),
        out_shape=jax.ShapeDtypeStruct((_HID, n_pad), jnp.float32),
    )(b_col, b_row, posp, posT, hT, W1a, w1pT, b1a_c, W1b, b1b_c)

    logits = pl.pallas_call(
        _l2_body,
        grid=(ni, nj),
        in_specs=[ispec, jspec_row, ispec_row, pi_spec, pT_spec,
                  pl.BlockSpec((_HID, _BJ), lambda i, j: (0, j)),
                  full(_HID + 2, _HID), full(_HID, 2), full(_HID, 1),
                  full(_HID, _HID), full(_HID, 1),
                  full(_HID, Wc.shape[1]), full(1, Wc.shape[1])],
        out_specs=pl.BlockSpec((_NG, Wc.shape[1]), lambda i, j: (0, 0)),
        out_shape=jax.ShapeDtypeStruct((_NG, Wc.shape[1]), jnp.float32),
        scratch_shapes=[pltpu.VMEM((_HID, _BI), jnp.float32),
                        pltpu.VMEM((_HID, _NG), jnp.float32)],
    )(b_col, b_row, b_row, posp, posT, x1T, W2a, w2pT, b2a_c, W2b,
      b2b_c, Wc, bc2)

    return logits


# speedup vs baseline: 3.8612x; 1.6383x over previous
import functools

import jax
import jax.numpy as jnp
from jax.experimental import pallas as pl
from jax.experimental.pallas import tpu as pltpu

_HID = 32
_NG = 8
_R2 = 625.0
_BI = 256
_BJ = 512
_JCH = 128
_NEG = -jnp.inf


def _accum_chunks(in_ch, relu_in, bi, bjr, pos_i, posT_j, fT_j, Wa, WapT, ba_c,
                  Wb, bb_c, acc):
    si = jnp.sum(pos_i * pos_i, axis=1, keepdims=True)
    Wa_h = Wa[:in_ch]
    bi_f = bi.astype(jnp.float32)
    n_i = pos_i.shape[0]
    wpb = WapT.astype(jnp.bfloat16).astype(jnp.float32)
    xi = pos_i[:, 0:1]
    yi = pos_i[:, 1:2]
    sh3 = (_HID, n_i, _JCH)
    wx3 = jax.lax.broadcast_in_dim(
        jnp.broadcast_to(wpb[:, 0:1], (_HID, _JCH)), sh3, (0, 2))
    wy3 = jax.lax.broadcast_in_dim(
        jnp.broadcast_to(wpb[:, 1:2], (_HID, _JCH)), sh3, (0, 2))
    for c in range(_BJ // _JCH):
        ls = slice(c * _JCH, (c + 1) * _JCH)
        pjt = posT_j[:, ls]
        fjt = fT_j[:, ls]
        if relu_in:
            fjt = jax.nn.relu(fjt)
        bj = bjr[:, ls].astype(jnp.float32)
        sj = jnp.sum(pjt * pjt, axis=0, keepdims=True)
        cross = jax.lax.dot_general(
            pos_i, pjt, (((1,), (0,)), ((), ())),
            preferred_element_type=jnp.float32)
        d2 = si + sj - 2.0 * cross
        pen = jnp.where((d2 <= _R2) & (bi_f == bj), 0.0, _NEG)
        ajh = jax.lax.dot_general(Wa_h, fjt, (((0,), (0,)), ((), ())),
                                  preferred_element_type=jnp.float32)
        ajh = ajh + ba_c
        dx = (pjt[0:1, :] - xi).astype(jnp.bfloat16).astype(jnp.float32)
        dy = (pjt[1:2, :] - yi).astype(jnp.bfloat16).astype(jnp.float32)
        pre3 = (jax.lax.broadcast_in_dim(ajh, sh3, (0, 2))
                + wx3 * jax.lax.broadcast_in_dim(dx, sh3, (1, 2))
                + wy3 * jax.lax.broadcast_in_dim(dy, sh3, (1, 2)))
        zT2 = jax.nn.relu(pre3).reshape(_HID, n_i * _JCH)
        mT = jax.lax.dot_general(Wb, zT2, (((0,), (0,)), ((), ())),
                                 preferred_element_type=jnp.float32) + bb_c
        mm = (mT.reshape(_HID, n_i, _JCH)
              + jax.lax.broadcast_in_dim(pen, (_HID, n_i, _JCH), (1, 2)))
        acc = jnp.maximum(acc, jnp.max(mm, axis=2))
    return acc


def _overlap(ck_i, ck_j):
    return ((jnp.min(ck_i) <= jnp.max(ck_j) + 3)
            & (jnp.min(ck_j) <= jnp.max(ck_i) + 3))


def _l1_body(bi_ref, bjr_ref, ck_i_ref, ck_j_ref, pos_i_ref, posT_ref,
             hT_ref, Wa_ref, WapT_ref, ba_ref, Wb_ref, bb_ref, out_ref):
    j = pl.program_id(1)

    @pl.when(j == 0)
    def _():
        out_ref[...] = jnp.full_like(out_ref, _NEG)

    bi = bi_ref[...]
    bjr = bjr_ref[...]

    @pl.when(_overlap(ck_i_ref[...], ck_j_ref[...]))
    def _():
        out_ref[...] = _accum_chunks(
            3, False, bi, bjr, pos_i_ref[...], posT_ref[...], hT_ref[...],
            Wa_ref[...], WapT_ref[...], ba_ref[...], Wb_ref[...], bb_ref[...],
            out_ref[...])


def _l2_body(bi_ref, bjr_ref, ck_i_ref, ck_j_ref, bir_ref, pos_i_ref,
             posT_ref, xT_ref, Wa_ref, WapT_ref, ba_ref, Wb_ref, bb_ref,
             Wc_ref, bc_ref, out_ref, acc_ref, g_ref):
    i = pl.program_id(0)
    j = pl.program_id(1)
    nj = pl.num_programs(1)

    @pl.when(j == 0)
    def _():
        acc_ref[...] = jnp.full_like(acc_ref, _NEG)

    @pl.when((i == 0) & (j == 0))
    def _():
        g_ref[...] = jnp.full_like(g_ref, _NEG)

    bi = bi_ref[...]
    bjr = bjr_ref[...]

    @pl.when(_overlap(ck_i_ref[...], ck_j_ref[...]))
    def _():
        acc_ref[...] = _accum_chunks(
            _HID, True, bi, bjr, pos_i_ref[...], posT_ref[...], xT_ref[...],
            Wa_ref[...], WapT_ref[...], ba_ref[...], Wb_ref[...], bb_ref[...],
            acc_ref[...])

    @pl.when(j == nj - 1)
    def _():
        r = jax.nn.relu(acc_ref[...])
        bir = bir_ref[...]
        reds = jnp.concatenate(
            [jnp.max(jnp.where(bir == g, r, _NEG), axis=1, keepdims=True)
             for g in range(_NG)], axis=1)
        g_new = jnp.maximum(g_ref[...], reds)
        g_ref[...] = g_new
        out_ref[...] = (jax.lax.dot_general(
            g_new, Wc_ref[...], (((0,), (0,)), ((), ())),
            preferred_element_type=jnp.float32) + bc_ref[...])


def kernel(h, pos, W1a, b1a, W1b, b1b, W2a, b2a, W2b, b2b, Wc, bc, batch):
    n = h.shape[0]
    n_pad = max(-(-n // _BJ) * _BJ, _BJ)
    pad = n_pad - n

    ky = jnp.clip((pos[:, 1] // 25.0).astype(jnp.int32), 0, 62)
    skey = batch.astype(jnp.int32) * 64 + ky
    perm = jnp.argsort(skey)
    h = h[perm]
    pos = pos[perm]
    batch = batch[perm]
    ckp = jnp.pad(skey[perm], (0, pad), constant_values=2 ** 20)
    ck_col = ckp.reshape(-1, 1)
    ck_row = ckp.reshape(1, -1)

    hT = jnp.pad(h, ((0, pad), (0, 0))).T
    posp = jnp.pad(pos, ((0, pad), (0, 0)), constant_values=1e6)
    batchp = jnp.pad(batch.astype(jnp.int32), (0, pad), constant_values=-1)
    b_col = batchp.reshape(-1, 1)
    b_row = batchp.reshape(1, -1)
    posT = posp.T
    b1a_c = b1a.reshape(-1, 1)
    w1pT = W1a[3:].T
    b1b_c = b1b.reshape(-1, 1)
    b2a_c = b2a.reshape(-1, 1)
    w2pT = W2a[_HID:].T
    b2b_c = b2b.reshape(-1, 1)
    bc2 = bc.reshape(1, -1)

    ni = n_pad // _BI
    nj = n_pad // _BJ
    ispec = pl.BlockSpec((_BI, 1), lambda i, j: (i, 0))
    jspec_row = pl.BlockSpec((1, _BJ), lambda i, j: (0, j))
    ck_ispec = pl.BlockSpec((_BI, 1), lambda i, j: (i, 0))
    ck_jspec = pl.BlockSpec((1, _BJ), lambda i, j: (0, j))
    ispec_row = pl.BlockSpec((1, _BI), lambda i, j: (0, i))
    pi_spec = pl.BlockSpec((_BI, 2), lambda i, j: (i, 0))
    pT_spec = pl.BlockSpec((2, _BJ), lambda i, j: (0, j))
    full = lambda r, c: pl.BlockSpec((r, c), lambda i, j: (0, 0))

    x1T = pl.pallas_call(
        _l1_body,
        grid=(ni, nj),
        in_specs=[ispec, jspec_row, ck_ispec, ck_jspec, pi_spec, pT_spec,
                  pl.BlockSpec((3, _BJ), lambda i, j: (0, j)),
                  full(5, _HID), full(_HID, 2), full(_HID, 1),
                  full(_HID, _HID), full(_HID, 1)],
        out_specs=pl.BlockSpec((_HID, _BI), lambda i, j: (0, i)),
        out_shape=jax.ShapeDtypeStruct((_HID, n_pad), jnp.float32),
    )(b_col, b_row, ck_col, ck_row, posp, posT, hT, W1a, w1pT, b1a_c,
      W1b, b1b_c)

    logits = pl.pallas_call(
        _l2_body,
        grid=(ni, nj),
        in_specs=[ispec, jspec_row, ck_ispec, ck_jspec, ispec_row, pi_spec,
                  pT_spec,
                  pl.BlockSpec((_HID, _BJ), lambda i, j: (0, j)),
                  full(_HID + 2, _HID), full(_HID, 2), full(_HID, 1),
                  full(_HID, _HID), full(_HID, 1),
                  full(_HID, Wc.shape[1]), full(1, Wc.shape[1])],
        out_specs=pl.BlockSpec((_NG, Wc.shape[1]), lambda i, j: (0, 0)),
        out_shape=jax.ShapeDtypeStruct((_NG, Wc.shape[1]), jnp.float32),
        scratch_shapes=[pltpu.VMEM((_HID, _BI), jnp.float32),
                        pltpu.VMEM((_HID, _NG), jnp.float32)],
    )(b_col, b_row, ck_col, ck_row, b_row, posp, posT, x1T, W2a, w2pT,
      b2a_c, W2b, b2b_c, Wc, bc2)

    return logits


# speedup vs baseline: 5.1790x; 1.3413x over previous
import functools

import jax
import jax.numpy as jnp
from jax.experimental import pallas as pl
from jax.experimental.pallas import tpu as pltpu

_HID = 32
_NG = 8
_R2 = 625.0
_BI = 256
_BJ = 512
_JCH = 128
_NEG = -jnp.inf


def _accum_chunks(in_ch, relu_in, bi, bjr, ck_i, ck_jr, pos_i, posT_j, fT_j,
                  Wa, WapT, ba_c, Wb, bb_c, acc_ref):
    si = jnp.sum(pos_i * pos_i, axis=1, keepdims=True)
    Wa_h = Wa[:in_ch]
    bi_f = bi.astype(jnp.float32)
    n_i = pos_i.shape[0]
    wpb = WapT.astype(jnp.bfloat16).astype(jnp.float32)
    xi = pos_i[:, 0:1]
    yi = pos_i[:, 1:2]
    sh3 = (_HID, n_i, _JCH)
    wx3 = jax.lax.broadcast_in_dim(
        jnp.broadcast_to(wpb[:, 0:1], (_HID, _JCH)), sh3, (0, 2))
    wy3 = jax.lax.broadcast_in_dim(
        jnp.broadcast_to(wpb[:, 1:2], (_HID, _JCH)), sh3, (0, 2))
    for c in range(_BJ // _JCH):
        ls = slice(c * _JCH, (c + 1) * _JCH)
        ckj = ck_jr[:, ls]
        ok = ((jnp.min(ck_i) <= jnp.max(ckj) + 2)
              & (jnp.min(ckj) <= jnp.max(ck_i) + 2))

        @pl.when(ok)
        def _(c=c, ls=ls):
            pjt = posT_j[:, ls]
            fjt = fT_j[:, ls]
            if relu_in:
                fjt = jax.nn.relu(fjt)
            bj = bjr[:, ls].astype(jnp.float32)
            sj = jnp.sum(pjt * pjt, axis=0, keepdims=True)
            cross = jax.lax.dot_general(
                pos_i, pjt, (((1,), (0,)), ((), ())),
                preferred_element_type=jnp.float32)
            d2 = si + sj - 2.0 * cross
            pen = jnp.where((d2 <= _R2) & (bi_f == bj), 0.0, _NEG)
            ajh = jax.lax.dot_general(Wa_h, fjt, (((0,), (0,)), ((), ())),
                                      preferred_element_type=jnp.float32)
            ajh = ajh + ba_c
            dx = (pjt[0:1, :] - xi).astype(jnp.bfloat16).astype(jnp.float32)
            dy = (pjt[1:2, :] - yi).astype(jnp.bfloat16).astype(jnp.float32)
            pre3 = (jax.lax.broadcast_in_dim(ajh, sh3, (0, 2))
                    + wx3 * jax.lax.broadcast_in_dim(dx, sh3, (1, 2))
                    + wy3 * jax.lax.broadcast_in_dim(dy, sh3, (1, 2)))
            zT2 = jax.nn.relu(pre3).reshape(_HID, n_i * _JCH)
            mT = jax.lax.dot_general(Wb, zT2, (((0,), (0,)), ((), ())),
                                     preferred_element_type=jnp.float32)
            mm = ((mT + bb_c).reshape(_HID, n_i, _JCH)
                  + jax.lax.broadcast_in_dim(pen, (_HID, n_i, _JCH), (1, 2)))
            acc_ref[...] = jnp.maximum(acc_ref[...], jnp.max(mm, axis=2))


def _overlap(ck_i, ck_j):
    return ((jnp.min(ck_i) <= jnp.max(ck_j) + 2)
            & (jnp.min(ck_j) <= jnp.max(ck_i) + 2))


def _l1_body(bi_ref, bjr_ref, ck_i_ref, ck_j_ref, pos_i_ref, posT_ref,
             hT_ref, Wa_ref, WapT_ref, ba_ref, Wb_ref, bb_ref, out_ref):
    j = pl.program_id(1)

    @pl.when(j == 0)
    def _():
        out_ref[...] = jnp.full_like(out_ref, _NEG)

    bi = bi_ref[...]
    bjr = bjr_ref[...]

    @pl.when(_overlap(ck_i_ref[...], ck_j_ref[...]))
    def _():
        _accum_chunks(
            3, False, bi, bjr, ck_i_ref[...], ck_j_ref[...], pos_i_ref[...],
            posT_ref[...], hT_ref[...], Wa_ref[...], WapT_ref[...],
            ba_ref[...], Wb_ref[...], bb_ref[...], out_ref)


def _l2_body(bi_ref, bjr_ref, ck_i_ref, ck_j_ref, bir_ref, pos_i_ref,
             posT_ref, xT_ref, Wa_ref, WapT_ref, ba_ref, Wb_ref, bb_ref,
             Wc_ref, bc_ref, out_ref, acc_ref, g_ref):
    i = pl.program_id(0)
    j = pl.program_id(1)
    nj = pl.num_programs(1)

    @pl.when(j == 0)
    def _():
        acc_ref[...] = jnp.full_like(acc_ref, _NEG)

    @pl.when((i == 0) & (j == 0))
    def _():
        g_ref[...] = jnp.full_like(g_ref, _NEG)

    bi = bi_ref[...]
    bjr = bjr_ref[...]

    @pl.when(_overlap(ck_i_ref[...], ck_j_ref[...]))
    def _():
        _accum_chunks(
            _HID, True, bi, bjr, ck_i_ref[...], ck_j_ref[...], pos_i_ref[...],
            posT_ref[...], xT_ref[...], Wa_ref[...], WapT_ref[...],
            ba_ref[...], Wb_ref[...], bb_ref[...], acc_ref)

    @pl.when(j == nj - 1)
    def _():
        r = jax.nn.relu(acc_ref[...])
        bir = bir_ref[...]
        reds = jnp.concatenate(
            [jnp.max(jnp.where(bir == g, r, _NEG), axis=1, keepdims=True)
             for g in range(_NG)], axis=1)
        g_new = jnp.maximum(g_ref[...], reds)
        g_ref[...] = g_new
        out_ref[...] = (jax.lax.dot_general(
            g_new, Wc_ref[...], (((0,), (0,)), ((), ())),
            preferred_element_type=jnp.float32) + bc_ref[...])


def kernel(h, pos, W1a, b1a, W1b, b1b, W2a, b2a, W2b, b2b, Wc, bc, batch):
    n = h.shape[0]
    n_pad = max(-(-n // _BJ) * _BJ, _BJ)
    pad = n_pad - n

    ky = jnp.clip((pos[:, 1] // 25.0).astype(jnp.int32), 0, 62)
    skey = batch.astype(jnp.int32) * 64 + ky
    perm = jnp.argsort(skey)
    h = h[perm]
    pos = pos[perm]
    batch = batch[perm]
    ckp = jnp.pad(skey[perm], (0, pad), constant_values=2 ** 20)
    ck_col = ckp.reshape(-1, 1)
    ck_row = ckp.reshape(1, -1)

    hT = jnp.pad(h, ((0, pad), (0, 0))).T
    posp = jnp.pad(pos, ((0, pad), (0, 0)), constant_values=1e6)
    batchp = jnp.pad(batch.astype(jnp.int32), (0, pad), constant_values=-1)
    b_col = batchp.reshape(-1, 1)
    b_row = batchp.reshape(1, -1)
    posT = posp.T
    b1a_c = b1a.reshape(-1, 1)
    w1pT = W1a[3:].T
    b1b_c = b1b.reshape(-1, 1)
    b2a_c = b2a.reshape(-1, 1)
    w2pT = W2a[_HID:].T
    b2b_c = b2b.reshape(-1, 1)
    bc2 = bc.reshape(1, -1)

    ni = n_pad // _BI
    nj = n_pad // _BJ
    ispec = pl.BlockSpec((_BI, 1), lambda i, j: (i, 0))
    jspec_row = pl.BlockSpec((1, _BJ), lambda i, j: (0, j))
    ck_ispec = pl.BlockSpec((_BI, 1), lambda i, j: (i, 0))
    ck_jspec = pl.BlockSpec((1, _BJ), lambda i, j: (0, j))
    ispec_row = pl.BlockSpec((1, _BI), lambda i, j: (0, i))
    pi_spec = pl.BlockSpec((_BI, 2), lambda i, j: (i, 0))
    pT_spec = pl.BlockSpec((2, _BJ), lambda i, j: (0, j))
    full = lambda r, c: pl.BlockSpec((r, c), lambda i, j: (0, 0))

    x1T = pl.pallas_call(
        _l1_body,
        grid=(ni, nj),
        in_specs=[ispec, jspec_row, ck_ispec, ck_jspec, pi_spec, pT_spec,
                  pl.BlockSpec((3, _BJ), lambda i, j: (0, j)),
                  full(5, _HID), full(_HID, 2), full(_HID, 1),
                  full(_HID, _HID), full(_HID, 1)],
        out_specs=pl.BlockSpec((_HID, _BI), lambda i, j: (0, i)),
        out_shape=jax.ShapeDtypeStruct((_HID, n_pad), jnp.float32),
    )(b_col, b_row, ck_col, ck_row, posp, posT, hT, W1a, w1pT, b1a_c,
      W1b, b1b_c)

    logits = pl.pallas_call(
        _l2_body,
        grid=(ni, nj),
        in_specs=[ispec, jspec_row, ck_ispec, ck_jspec, ispec_row, pi_spec,
                  pT_spec,
                  pl.BlockSpec((_HID, _BJ), lambda i, j: (0, j)),
                  full(_HID + 2, _HID), full(_HID, 2), full(_HID, 1),
                  full(_HID, _HID), full(_HID, 1),
                  full(_HID, Wc.shape[1]), full(1, Wc.shape[1])],
        out_specs=pl.BlockSpec((_NG, Wc.shape[1]), lambda i, j: (0, 0)),
        out_shape=jax.ShapeDtypeStruct((_NG, Wc.shape[1]), jnp.float32),
        scratch_shapes=[pltpu.VMEM((_HID, _BI), jnp.float32),
                        pltpu.VMEM((_HID, _NG), jnp.float32)],
    )(b_col, b_row, ck_col, ck_row, b_row, posp, posT, x1T, W2a, w2pT,
      b2a_c, W2b, b2b_c, Wc, bc2)

    return logits
